# baseline (speedup 1.0000x reference)
import functools

import jax
import jax.numpy as jnp
from jax import lax
from jax.experimental import pallas as pl
from jax.experimental.pallas import tpu as pltpu
from jax.experimental.pallas import tpu_sc as plsc

NUM_KEYS = 13
EMBED_DIM = 64
BATCH = 16384

NC, NS = 2, 16
NW = NC * NS
BPW = BATCH // NW
CHUNK = 128
NCHUNK = BPW // CHUNK


def _proj_body(emb_ref, w_ref, b_ref, out_ref):
    out_ref[...] = lax.dot_general(
        emb_ref[...], w_ref[...],
        dimension_numbers=(((1,), (1,)), ((), ())),
        preferred_element_type=jnp.float32,
    ) + b_ref[...]


def _project_table(emb_table, W, b):
    return pl.pallas_call(
        _proj_body,
        out_shape=jax.ShapeDtypeStruct((NUM_KEYS, EMBED_DIM), jnp.float32),
    )(emb_table, W, b.reshape(1, EMBED_DIM))


def _gather_body(table_hbm, idx_hbm, out_hbm, idx_v, rows_v, sem):
    wid = lax.axis_index("s") * NC + lax.axis_index("c")
    pltpu.sync_copy(idx_hbm.at[wid], idx_v)
    copies = [
        pltpu.async_copy(
            table_hbm.at[idx_v.at[j]],
            rows_v.at[pl.ds(j * CHUNK, CHUNK)],
            sem,
        )
        for j in range(NCHUNK)
    ]
    for c in copies:
        c.wait()
    pltpu.sync_copy(rows_v, out_hbm.at[pl.ds(wid * BPW, BPW)])


_gather = pl.kernel(
    _gather_body,
    out_type=jax.ShapeDtypeStruct((BATCH, EMBED_DIM), jnp.float32),
    mesh=plsc.VectorSubcoreMesh(
        core_axis_name="c", subcore_axis_name="s",
        num_cores=NC, num_subcores=NS,
    ),
    scratch_types=[
        pltpu.VMEM((NCHUNK, CHUNK), jnp.int32),
        pltpu.VMEM((BPW, EMBED_DIM), jnp.float32),
        pltpu.SemaphoreType.DMA,
    ],
    compiler_params=pltpu.CompilerParams(use_tc_tiling_on_sc=False),
)


def kernel(key_idx, emb_table, W, b):
    proj = _project_table(emb_table, W, b)
    idx = key_idx.astype(jnp.int32).reshape(NW, NCHUNK, CHUNK)
    out = _gather(proj, idx)
    return out[:, None, :]


# speedup vs baseline: 1.9791x; 1.9791x over previous
import functools

import jax
import jax.numpy as jnp
from jax import lax
from jax.experimental import pallas as pl
from jax.experimental.pallas import tpu as pltpu
from jax.experimental.pallas import tpu_sc as plsc

NUM_KEYS = 13
EMBED_DIM = 64
BATCH = 16384

NC, NS = 2, 16
NW = NC * NS
BPW = BATCH // NW
LANES = 16
DCHUNKS = EMBED_DIM // LANES


def _proj_body(emb_ref, w_ref, b_ref, out_ref):
    out_ref[...] = lax.dot_general(
        emb_ref[...], w_ref[...],
        dimension_numbers=(((1,), (1,)), ((), ())),
        preferred_element_type=jnp.float32,
    ) + b_ref[...]


def _project_table(emb_table, W, b):
    return pl.pallas_call(
        _proj_body,
        out_shape=jax.ShapeDtypeStruct((NUM_KEYS, EMBED_DIM), jnp.float32),
    )(emb_table, W, b.reshape(1, EMBED_DIM))


def _gather_body(table_hbm, idx_hbm, out_hbm, table_v, idx_v, rows_v, sem):
    wid = lax.axis_index("s") * NC + lax.axis_index("c")
    pltpu.async_copy(table_hbm, table_v, sem).wait()
    pltpu.sync_copy(idx_hbm.at[wid], idx_v)

    def body(g, carry):
        vi = idx_v[pl.ds(g * LANES, LANES)]
        for j in range(LANES):
            r = vi[j]
            i = g * LANES + j
            for d in range(DCHUNKS):
                rows_v[i, pl.ds(d * LANES, LANES)] = (
                    table_v[r, pl.ds(d * LANES, LANES)])
        return carry

    lax.fori_loop(0, BPW // LANES, body, 0)
    pltpu.sync_copy(rows_v, out_hbm.at[pl.ds(wid * BPW, BPW)])


_gather = pl.kernel(
    _gather_body,
    out_type=jax.ShapeDtypeStruct((BATCH, EMBED_DIM), jnp.float32),
    mesh=plsc.VectorSubcoreMesh(
        core_axis_name="c", subcore_axis_name="s",
        num_cores=NC, num_subcores=NS,
    ),
    scratch_types=[
        pltpu.VMEM((NUM_KEYS, EMBED_DIM), jnp.float32),
        pltpu.VMEM((BPW,), jnp.int32),
        pltpu.VMEM((BPW, EMBED_DIM), jnp.float32),
        pltpu.SemaphoreType.DMA,
    ],
    compiler_params=pltpu.CompilerParams(use_tc_tiling_on_sc=False),
)


def kernel(key_idx, emb_table, W, b):
    proj = _project_table(emb_table, W, b)
    idx = key_idx.astype(jnp.int32).reshape(NW, BPW)
    out = _gather(proj, idx)
    return out[:, None, :]


# speedup vs baseline: 2.3281x; 1.1764x over previous
import functools

import jax
import jax.numpy as jnp
from jax import lax
from jax.experimental import pallas as pl
from jax.experimental.pallas import tpu as pltpu
from jax.experimental.pallas import tpu_sc as plsc

NUM_KEYS = 13
EMBED_DIM = 64
BATCH = 16384

NC, NS = 2, 16
NW = NC * NS
BPW = BATCH // NW
LANES = 16
DCHUNKS = EMBED_DIM // LANES


def _proj_body(emb_ref, w_ref, b_ref, out_ref):
    out_ref[...] = lax.dot_general(
        emb_ref[...], w_ref[...],
        dimension_numbers=(((1,), (1,)), ((), ())),
        preferred_element_type=jnp.float32,
    ) + b_ref[...]


def _project_table(emb_table, W, b):
    return pl.pallas_call(
        _proj_body,
        out_shape=jax.ShapeDtypeStruct((NUM_KEYS, EMBED_DIM), jnp.float32),
    )(emb_table, W, b.reshape(1, EMBED_DIM))


def _gather_body(table_hbm, idx_hbm, out_hbm, table_v, idx_v, rows_v, sem):
    wid = lax.axis_index("s") * NC + lax.axis_index("c")
    pltpu.async_copy(table_hbm, table_v, sem).wait()
    pltpu.sync_copy(idx_hbm.at[pl.ds(wid * BPW, BPW)], idx_v)

    def body(g, carry):
        vi = idx_v[pl.ds(g * LANES, LANES)]
        for j in range(LANES):
            r = vi[j]
            i = g * LANES + j
            for d in range(DCHUNKS):
                rows_v[i, pl.ds(d * LANES, LANES)] = (
                    table_v[r, pl.ds(d * LANES, LANES)])
        return carry

    lax.fori_loop(0, BPW // LANES, body, 0)
    pltpu.sync_copy(rows_v, out_hbm.at[pl.ds(wid * BPW, BPW)])


_gather = pl.kernel(
    _gather_body,
    out_type=jax.ShapeDtypeStruct((BATCH, EMBED_DIM), jnp.float32),
    mesh=plsc.VectorSubcoreMesh(
        core_axis_name="c", subcore_axis_name="s",
        num_cores=NC, num_subcores=NS,
    ),
    scratch_types=[
        pltpu.VMEM((NUM_KEYS, EMBED_DIM), jnp.float32),
        pltpu.VMEM((BPW,), jnp.int32),
        pltpu.VMEM((BPW, EMBED_DIM), jnp.float32),
        pltpu.SemaphoreType.DMA,
    ],
    compiler_params=pltpu.CompilerParams(use_tc_tiling_on_sc=True),
)


def kernel(key_idx, emb_table, W, b):
    proj = _project_table(emb_table, W, b)
    idx = key_idx.astype(jnp.int32)
    out = _gather(proj, idx)
    return out[:, None, :]
